# baseline (speedup 1.0000x reference)
import functools

import jax
import jax.numpy as jnp
from jax import lax
from jax.experimental import pallas as pl
from jax.experimental.pallas import tpu as pltpu
from jax.experimental.pallas import tpu_sc as plsc

NP = 10240
NW = 32
CHUNK = 64
NR = 3
G = 10
EPW = 160 * CHUNK
EPAD = NW * EPW
T = EPW // CHUNK


def _make_agg(with_cnt=False):
    rpt = NP // 16

    mesh = plsc.VectorSubcoreMesh(core_axis_name="c", subcore_axis_name="s")
    out_type = [jax.ShapeDtypeStruct((2, NP, 128), jnp.float32)]
    scratch = (
        [pltpu.VMEM((CHUNK,), jnp.int32) for _ in range(G)]
        + [pltpu.VMEM((CHUNK,), jnp.int32) for _ in range(G)]
        + [pltpu.VMEM((CHUNK, 128), jnp.float32) for _ in range(NR)]
        + [pltpu.SemaphoreType.DMA for _ in range(G)]
        + [pltpu.SemaphoreType.DMA for _ in range(NR)]
        + [pltpu.VMEM_SHARED((NP, 128), jnp.float32)]
    )
    if with_cnt:
        out_type.append(jax.ShapeDtypeStruct((2, NP, 128), jnp.float32))
        scratch.append(pltpu.VMEM((CHUNK, 128), jnp.float32))

    def body(*refs):
        if with_cnt:
            (y_hbm, src_hbm, dst_hbm, zf_hbm, ones_hbm,
             out_s, out_c, *scr) = refs
        else:
            (y_hbm, src_hbm, dst_hbm, zf_hbm, out_s, *scr) = refs
        scur = scr[0:G]
        dcur = scr[G:2 * G]
        rows = scr[2 * G:2 * G + NR]
        sem_i = scr[2 * G + NR:3 * G + NR]
        sem_g = scr[3 * G + NR:3 * G + 2 * NR]
        acc = scr[3 * G + 2 * NR]
        cid = lax.axis_index("c")
        sid = lax.axis_index("s")
        wid = sid * 2 + cid
        r0 = sid * rpt
        base0 = wid * EPW

        def fire_idx(c, j, with_src=True):
            ds = []
            if with_src:
                ds.append(pltpu.async_copy(
                    src_hbm.at[pl.ds(base0 + c * CHUNK, CHUNK)],
                    scur[j], sem_i[j]))
            ds.append(pltpu.async_copy(
                dst_hbm.at[pl.ds(base0 + c * CHUNK, CHUNK)],
                dcur[j], sem_i[j]))
            return ds

        def zero_acc():
            pltpu.sync_copy(zf_hbm.at[pl.ds(r0, rpt)], acc.at[pl.ds(r0, rpt)])

        if with_cnt:
            ones_v = scr[3 * G + 2 * NR + 1]
            zero_acc()
            pltpu.sync_copy(ones_hbm, ones_v)
            plsc.subcore_barrier()
            @pl.loop(0, T, step=G)
            def _cnt(t0):
                idescs = [fire_idx(t0 + j, j, with_src=False) for j in range(G)]
                for j in range(G):
                    for d in idescs[j]:
                        d.wait()
                    pltpu.sync_copy(ones_v, acc.at[dcur[j]], add=True)

            plsc.subcore_barrier()
            pltpu.sync_copy(acc.at[pl.ds(r0, rpt)],
                            out_c.at[cid, pl.ds(r0, rpt)])

        zero_acc()
        plsc.subcore_barrier()

        @pl.loop(0, T, step=G)
        def _steady(t0):
            idescs = [fire_idx(t0 + j, j) for j in range(G)]
            gdescs = []
            for j in range(G):
                if j >= NR:
                    gdescs[j - NR].wait()
                    pltpu.sync_copy(rows[(j - NR) % NR],
                                    acc.at[dcur[j - NR]], add=True)
                for d in idescs[j]:
                    d.wait()
                gdescs.append(
                    pltpu.async_copy(y_hbm.at[scur[j]], rows[j % NR],
                                     sem_g[j % NR]))
            for j in range(G - NR, G):
                gdescs[j].wait()
                pltpu.sync_copy(rows[j % NR], acc.at[dcur[j]], add=True)

        plsc.subcore_barrier()
        pltpu.sync_copy(acc.at[pl.ds(r0, rpt)], out_s.at[cid, pl.ds(r0, rpt)])

    return pl.kernel(body, out_type=out_type, mesh=mesh, scratch_types=scratch)


def _lrelu(v):
    return jnp.where(v >= 0.0, v, 0.01 * v)


def _mm(a, b):
    return jnp.dot(a.astype(jnp.bfloat16), b.astype(jnp.bfloat16),
                   preferred_element_type=jnp.float32)


def _bs(shape, imap):
    return pl.BlockSpec(shape, imap)


ROWBLK = 512

_ROWMAP = lambda i: (i, 0)
_PARTMAP = lambda i: (0, i, 0)
_FIXMAP = lambda i: (0, 0)

_SP128 = _bs((2, ROWBLK, 128), _PARTMAP)
_H = _bs((ROWBLK, 128), _ROWMAP)
_H16 = _bs((ROWBLK, 16), _ROWMAP)
_H64 = _bs((ROWBLK, 64), _ROWMAP)
_W128 = _bs((128, 128), _FIXMAP)
_W64 = _bs((128, 64), _FIXMAP)
_W64_128 = _bs((64, 128), _FIXMAP)
_W16 = _bs((128, 16), _FIXMAP)
_B128 = _bs((1, 128), _FIXMAP)
_B64 = _bs((1, 64), _FIXMAP)
_B16 = _bs((1, 16), _FIXMAP)

_GRID = (NP // ROWBLK,)


def _call(body, in_specs, out_specs, out_shapes):
    return pl.pallas_call(
        body,
        grid=_GRID,
        in_specs=in_specs,
        out_specs=out_specs,
        out_shape=out_shapes,
    )


def _l1_body(sp, cp, h, wl, bl, wr, o, dv):
    cnt = cp[0, :, 0:16] + cp[1, :, 0:16]
    dinv = 1.0 / jnp.maximum(cnt, 1.0)
    agg = (sp[0] + sp[1]) * dinv[:, 0:1]
    o[...] = jnp.tanh(_mm(agg, wl[...]) + bl[...] + _mm(h[...], wr[...]))
    dv[...] = dinv


def _mid_body(act, sp, dv, h, wl, bl, wr, o):
    agg = (sp[0] + sp[1]) * dv[:, 0:1]
    o[...] = act(_mm(agg, wl[...]) + bl[...] + _mm(h[...], wr[...]))


def _l3_body(sp, dv, h, wl, bl, wr, wt, bt, wd1, bd1, mu_o, o0_o):
    agg = (sp[0] + sp[1]) * dv[:, 0:1]
    h3 = jnp.tanh(_mm(agg, wl[...]) + bl[...] + _mm(h[...], wr[...]))
    mu = _mm(h3, wt[...]) + bt[...]
    mu_o[...] = mu
    o0_o[...] = _lrelu(_mm(mu, wd1[...]) + bd1[...])


def _l6_body(sp, dv, h, wl6, bl6, wr6, z_o):
    agg = (sp[0] + sp[1]) * dv[:, 0:1]
    z_o[...] = _mm(agg, wl6[...]) + bl6[...] + _mm(h[...], wr6[...])


def kernel(x, edge_index, Wl1, bl1, Wr1, Wl2, bl2, Wr2, Wl3, bl3, Wr3,
           Wt, bt, Wd1, bd1, Wl4, bl4, Wr4, Wl5, bl5, Wr5, Wl6, bl6, Wr6):
    n, d = x.shape
    n_edges = edge_index.shape[1]
    f32 = jnp.float32

    xp = jnp.zeros((NP, d), f32).at[:n].set(x)
    epad = jnp.full((2, EPAD + 3 * CHUNK - n_edges), NP - 1, jnp.int32)
    ei = jnp.concatenate([edge_index.astype(jnp.int32), epad], axis=1)
    srcv = ei[0]
    dstv = ei[1]
    z128 = jnp.zeros((NP, 128), f32)
    onesc = jnp.ones((CHUNK, 128), f32)
    bl1r, bl2r, bl3r = bl1.reshape(1, 128), bl2.reshape(1, 128), bl3.reshape(1, 128)
    bl4r, bl5r = bl4.reshape(1, 128), bl5.reshape(1, 128)
    btr, bd1r = bt.reshape(1, 64), bd1.reshape(1, 128)
    wl6p = jnp.zeros((128, 16), f32).at[:, :3].set(Wl6)
    wr6p = jnp.zeros((128, 16), f32).at[:, :3].set(Wr6)
    bl6p = jnp.zeros((1, 16), f32).at[0, :3].set(bl6)

    agg = _make_agg()

    s1p, c1p = _make_agg(with_cnt=True)(xp, srcv, dstv, z128, onesc)
    h1, dv = _call(
        _l1_body,
        [_SP128, _SP128, _H, _W128, _B128, _W128],
        [_H, _H16],
        [jax.ShapeDtypeStruct((NP, 128), f32), jax.ShapeDtypeStruct((NP, 16), f32)],
    )(s1p, c1p, xp, Wl1, bl1r, Wr1)

    (s2p,) = agg(h1, srcv, dstv, z128)
    h2 = _call(
        functools.partial(_mid_body, jnp.tanh),
        [_SP128, _H16, _H, _W128, _B128, _W128],
        _H,
        jax.ShapeDtypeStruct((NP, 128), f32),
    )(s2p, dv, h1, Wl2, bl2r, Wr2)

    (s3p,) = agg(h2, srcv, dstv, z128)
    mu, o0 = _call(
        _l3_body,
        [_SP128, _H16, _H, _W128, _B128, _W128, _W64, _B64, _W64_128, _B128],
        [_H64, _H],
        [jax.ShapeDtypeStruct((NP, 64), f32), jax.ShapeDtypeStruct((NP, 128), f32)],
    )(s3p, dv, h2, Wl3, bl3r, Wr3, Wt, btr, Wd1, bd1r)

    (s4p,) = agg(o0, srcv, dstv, z128)
    o1 = _call(
        functools.partial(_mid_body, _lrelu),
        [_SP128, _H16, _H, _W128, _B128, _W128],
        _H,
        jax.ShapeDtypeStruct((NP, 128), f32),
    )(s4p, dv, o0, Wl4, bl4r, Wr4)

    (s5p,) = agg(o1, srcv, dstv, z128)
    o2 = _call(
        functools.partial(_mid_body, jnp.tanh),
        [_SP128, _H16, _H, _W128, _B128, _W128],
        _H,
        jax.ShapeDtypeStruct((NP, 128), f32),
    )(s5p, dv, o1, Wl5, bl5r, Wr5)

    (s6p,) = agg(o2, srcv, dstv, z128)
    z2 = _call(
        _l6_body,
        [_SP128, _H16, _H, _W16, _B16, _W16],
        _H16,
        jax.ShapeDtypeStruct((NP, 16), f32),
    )(s6p, dv, o2, wl6p, bl6p, wr6p)

    mu_n = mu[:n]
    return (z2[:n, :2], z2[:n, 2], mu_n, mu_n)


# speedup vs baseline: 1.0035x; 1.0035x over previous
import functools

import jax
import jax.numpy as jnp
from jax import lax
from jax.experimental import pallas as pl
from jax.experimental.pallas import tpu as pltpu
from jax.experimental.pallas import tpu_sc as plsc

NP = 10240
NW = 32
CHUNK = 128
NR = 2
G = 10
EPW = 80 * CHUNK
EPAD = NW * EPW
T = EPW // CHUNK


def _make_agg(with_cnt=False):
    rpt = NP // 16

    mesh = plsc.VectorSubcoreMesh(core_axis_name="c", subcore_axis_name="s")
    out_type = [jax.ShapeDtypeStruct((2, NP, 128), jnp.float32)]
    scratch = (
        [pltpu.VMEM((CHUNK,), jnp.int32) for _ in range(G)]
        + [pltpu.VMEM((CHUNK,), jnp.int32) for _ in range(G)]
        + [pltpu.VMEM((CHUNK, 128), jnp.float32) for _ in range(NR)]
        + [pltpu.SemaphoreType.DMA for _ in range(G)]
        + [pltpu.SemaphoreType.DMA for _ in range(NR)]
        + [pltpu.VMEM_SHARED((NP, 128), jnp.float32)]
    )
    if with_cnt:
        out_type.append(jax.ShapeDtypeStruct((2, NP, 128), jnp.float32))

    def body(*refs):
        if with_cnt:
            (y_hbm, src_hbm, dst_hbm, zf_hbm, ones_hbm,
             out_s, out_c, *scr) = refs
        else:
            (y_hbm, src_hbm, dst_hbm, zf_hbm, out_s, *scr) = refs
        scur = scr[0:G]
        dcur = scr[G:2 * G]
        rows = scr[2 * G:2 * G + NR]
        sem_i = scr[2 * G + NR:3 * G + NR]
        sem_g = scr[3 * G + NR:3 * G + 2 * NR]
        acc = scr[3 * G + 2 * NR]
        cid = lax.axis_index("c")
        sid = lax.axis_index("s")
        wid = sid * 2 + cid
        r0 = sid * rpt
        base0 = wid * EPW

        def fire_idx(c, j, with_src=True):
            ds = []
            if with_src:
                ds.append(pltpu.async_copy(
                    src_hbm.at[pl.ds(base0 + c * CHUNK, CHUNK)],
                    scur[j], sem_i[j]))
            ds.append(pltpu.async_copy(
                dst_hbm.at[pl.ds(base0 + c * CHUNK, CHUNK)],
                dcur[j], sem_i[j]))
            return ds

        def zero_acc():
            pltpu.sync_copy(zf_hbm.at[pl.ds(r0, rpt)], acc.at[pl.ds(r0, rpt)])

        if with_cnt:
            ones_v = rows[0]
            zero_acc()
            pltpu.sync_copy(ones_hbm, ones_v)
            plsc.subcore_barrier()
            @pl.loop(0, T, step=G)
            def _cnt(t0):
                idescs = [fire_idx(t0 + j, j, with_src=False) for j in range(G)]
                for j in range(G):
                    for d in idescs[j]:
                        d.wait()
                    pltpu.sync_copy(ones_v, acc.at[dcur[j]], add=True)

            plsc.subcore_barrier()
            pltpu.sync_copy(acc.at[pl.ds(r0, rpt)],
                            out_c.at[cid, pl.ds(r0, rpt)])

        zero_acc()
        plsc.subcore_barrier()

        @pl.loop(0, T, step=G)
        def _steady(t0):
            idescs = [fire_idx(t0 + j, j) for j in range(G)]
            gdescs = []
            for j in range(G):
                if j >= NR:
                    gdescs[j - NR].wait()
                    pltpu.sync_copy(rows[(j - NR) % NR],
                                    acc.at[dcur[j - NR]], add=True)
                for d in idescs[j]:
                    d.wait()
                gdescs.append(
                    pltpu.async_copy(y_hbm.at[scur[j]], rows[j % NR],
                                     sem_g[j % NR]))
            for j in range(G - NR, G):
                gdescs[j].wait()
                pltpu.sync_copy(rows[j % NR], acc.at[dcur[j]], add=True)

        plsc.subcore_barrier()
        pltpu.sync_copy(acc.at[pl.ds(r0, rpt)], out_s.at[cid, pl.ds(r0, rpt)])

    return pl.kernel(body, out_type=out_type, mesh=mesh, scratch_types=scratch)


def _lrelu(v):
    return jnp.where(v >= 0.0, v, 0.01 * v)


def _mm(a, b):
    return jnp.dot(a.astype(jnp.bfloat16), b.astype(jnp.bfloat16),
                   preferred_element_type=jnp.float32)


def _bs(shape, imap):
    return pl.BlockSpec(shape, imap)


ROWBLK = 512

_ROWMAP = lambda i: (i, 0)
_PARTMAP = lambda i: (0, i, 0)
_FIXMAP = lambda i: (0, 0)

_SP128 = _bs((2, ROWBLK, 128), _PARTMAP)
_H = _bs((ROWBLK, 128), _ROWMAP)
_H16 = _bs((ROWBLK, 16), _ROWMAP)
_H64 = _bs((ROWBLK, 64), _ROWMAP)
_W128 = _bs((128, 128), _FIXMAP)
_W64 = _bs((128, 64), _FIXMAP)
_W64_128 = _bs((64, 128), _FIXMAP)
_W16 = _bs((128, 16), _FIXMAP)
_B128 = _bs((1, 128), _FIXMAP)
_B64 = _bs((1, 64), _FIXMAP)
_B16 = _bs((1, 16), _FIXMAP)

_GRID = (NP // ROWBLK,)


def _call(body, in_specs, out_specs, out_shapes):
    return pl.pallas_call(
        body,
        grid=_GRID,
        in_specs=in_specs,
        out_specs=out_specs,
        out_shape=out_shapes,
    )


def _l1_body(sp, cp, h, wl, bl, wr, o, dv):
    cnt = cp[0, :, 0:16] + cp[1, :, 0:16]
    dinv = 1.0 / jnp.maximum(cnt, 1.0)
    agg = (sp[0] + sp[1]) * dinv[:, 0:1]
    o[...] = jnp.tanh(_mm(agg, wl[...]) + bl[...] + _mm(h[...], wr[...]))
    dv[...] = dinv


def _mid_body(act, sp, dv, h, wl, bl, wr, o):
    agg = (sp[0] + sp[1]) * dv[:, 0:1]
    o[...] = act(_mm(agg, wl[...]) + bl[...] + _mm(h[...], wr[...]))


def _l3_body(sp, dv, h, wl, bl, wr, wt, bt, wd1, bd1, mu_o, o0_o):
    agg = (sp[0] + sp[1]) * dv[:, 0:1]
    h3 = jnp.tanh(_mm(agg, wl[...]) + bl[...] + _mm(h[...], wr[...]))
    mu = _mm(h3, wt[...]) + bt[...]
    mu_o[...] = mu
    o0_o[...] = _lrelu(_mm(mu, wd1[...]) + bd1[...])


def _l6_body(sp, dv, h, wl6, bl6, wr6, z_o):
    agg = (sp[0] + sp[1]) * dv[:, 0:1]
    z_o[...] = _mm(agg, wl6[...]) + bl6[...] + _mm(h[...], wr6[...])


def kernel(x, edge_index, Wl1, bl1, Wr1, Wl2, bl2, Wr2, Wl3, bl3, Wr3,
           Wt, bt, Wd1, bd1, Wl4, bl4, Wr4, Wl5, bl5, Wr5, Wl6, bl6, Wr6):
    n, d = x.shape
    n_edges = edge_index.shape[1]
    f32 = jnp.float32

    xp = jnp.zeros((NP, d), f32).at[:n].set(x)
    epad = jnp.full((2, EPAD + 3 * CHUNK - n_edges), NP - 1, jnp.int32)
    ei = jnp.concatenate([edge_index.astype(jnp.int32), epad], axis=1)
    srcv = ei[0]
    dstv = ei[1]
    z128 = jnp.zeros((NP, 128), f32)
    onesc = jnp.ones((CHUNK, 128), f32)
    bl1r, bl2r, bl3r = bl1.reshape(1, 128), bl2.reshape(1, 128), bl3.reshape(1, 128)
    bl4r, bl5r = bl4.reshape(1, 128), bl5.reshape(1, 128)
    btr, bd1r = bt.reshape(1, 64), bd1.reshape(1, 128)
    wl6p = jnp.zeros((128, 16), f32).at[:, :3].set(Wl6)
    wr6p = jnp.zeros((128, 16), f32).at[:, :3].set(Wr6)
    bl6p = jnp.zeros((1, 16), f32).at[0, :3].set(bl6)

    agg = _make_agg()

    s1p, c1p = _make_agg(with_cnt=True)(xp, srcv, dstv, z128, onesc)
    h1, dv = _call(
        _l1_body,
        [_SP128, _SP128, _H, _W128, _B128, _W128],
        [_H, _H16],
        [jax.ShapeDtypeStruct((NP, 128), f32), jax.ShapeDtypeStruct((NP, 16), f32)],
    )(s1p, c1p, xp, Wl1, bl1r, Wr1)

    (s2p,) = agg(h1, srcv, dstv, z128)
    h2 = _call(
        functools.partial(_mid_body, jnp.tanh),
        [_SP128, _H16, _H, _W128, _B128, _W128],
        _H,
        jax.ShapeDtypeStruct((NP, 128), f32),
    )(s2p, dv, h1, Wl2, bl2r, Wr2)

    (s3p,) = agg(h2, srcv, dstv, z128)
    mu, o0 = _call(
        _l3_body,
        [_SP128, _H16, _H, _W128, _B128, _W128, _W64, _B64, _W64_128, _B128],
        [_H64, _H],
        [jax.ShapeDtypeStruct((NP, 64), f32), jax.ShapeDtypeStruct((NP, 128), f32)],
    )(s3p, dv, h2, Wl3, bl3r, Wr3, Wt, btr, Wd1, bd1r)

    (s4p,) = agg(o0, srcv, dstv, z128)
    o1 = _call(
        functools.partial(_mid_body, _lrelu),
        [_SP128, _H16, _H, _W128, _B128, _W128],
        _H,
        jax.ShapeDtypeStruct((NP, 128), f32),
    )(s4p, dv, o0, Wl4, bl4r, Wr4)

    (s5p,) = agg(o1, srcv, dstv, z128)
    o2 = _call(
        functools.partial(_mid_body, jnp.tanh),
        [_SP128, _H16, _H, _W128, _B128, _W128],
        _H,
        jax.ShapeDtypeStruct((NP, 128), f32),
    )(s5p, dv, o1, Wl5, bl5r, Wr5)

    (s6p,) = agg(o2, srcv, dstv, z128)
    z2 = _call(
        _l6_body,
        [_SP128, _H16, _H, _W16, _B16, _W16],
        _H16,
        jax.ShapeDtypeStruct((NP, 16), f32),
    )(s6p, dv, o2, wl6p, bl6p, wr6p)

    mu_n = mu[:n]
    return (z2[:n, :2], z2[:n, 2], mu_n, mu_n)


# speedup vs baseline: 1.2139x; 1.2096x over previous
import functools

import jax
import jax.numpy as jnp
from jax import lax
from jax.experimental import pallas as pl
from jax.experimental.pallas import tpu as pltpu
from jax.experimental.pallas import tpu_sc as plsc

NP = 10240
NW = 32
CHUNK = 96
NR = 2
EPW = 106 * CHUNK
EPAD = NW * EPW
T = EPW // CHUNK


def _make_agg(with_cnt=False):
    rpt = NP // 16

    mesh = plsc.VectorSubcoreMesh(core_axis_name="c", subcore_axis_name="s")
    out_type = [jax.ShapeDtypeStruct((2, NP, 128), jnp.float32)]
    scratch = (
        [pltpu.VMEM((EPW,), jnp.int32)]
        + [pltpu.VMEM((T, 1, CHUNK), jnp.int32)]
        + [pltpu.VMEM((CHUNK, 128), jnp.float32) for _ in range(NR)]
        + [pltpu.SemaphoreType.DMA for _ in range(NR)]
        + [pltpu.VMEM_SHARED((NP, 128), jnp.float32)]
    )
    if with_cnt:
        out_type.append(jax.ShapeDtypeStruct((2, NP, 128), jnp.float32))

    def body(*refs):
        if with_cnt:
            (y_hbm, src_hbm, dst_hbm, zf_hbm, ones_hbm,
             out_s, out_c, *scr) = refs
        else:
            (y_hbm, src_hbm, dst_hbm, zf_hbm, out_s, *scr) = refs
        sidx, didx = scr[0], scr[1]
        rows = scr[2:2 + NR]
        sem_g = scr[2 + NR:2 + 2 * NR]
        acc = scr[2 + 2 * NR]
        cid = lax.axis_index("c")
        sid = lax.axis_index("s")
        wid = sid * 2 + cid
        r0 = sid * rpt

        pltpu.sync_copy(src_hbm.at[pl.ds(wid * EPW, EPW)], sidx)
        pltpu.sync_copy(dst_hbm.at[wid], didx)

        def gidx(c):
            return sidx.at[pl.ds(c * CHUNK, CHUNK)]

        def fire_gather(c, j):
            return pltpu.async_copy(y_hbm.at[gidx(c)], rows[j], sem_g[j])

        def wait_gather(j):
            pltpu.make_async_copy(y_hbm.at[gidx(0)], rows[j], sem_g[j]).wait()

        def scatter(c, j, src_v):
            pltpu.sync_copy(src_v, acc.at[didx.at[c, 0]], add=True)

        def zero_acc():
            pltpu.sync_copy(zf_hbm.at[pl.ds(r0, rpt)], acc.at[pl.ds(r0, rpt)])

        if with_cnt:
            ones_v = rows[0]
            zero_acc()
            pltpu.sync_copy(ones_hbm, ones_v)
            plsc.subcore_barrier()

            @pl.loop(0, T)
            def _cnt(c):
                scatter(c, 0, ones_v)

            plsc.subcore_barrier()
            pltpu.sync_copy(acc.at[pl.ds(r0, rpt)],
                            out_c.at[cid, pl.ds(r0, rpt)])

        zero_acc()
        plsc.subcore_barrier()

        for j in range(NR):
            fire_gather(j, j)

        @pl.loop(0, T - NR, step=NR)
        def _steady(t0):
            for j in range(NR):
                c = t0 + j
                wait_gather(j)
                scatter(c, j, rows[j])
                fire_gather(c + NR, j)

        for j in range(NR):
            wait_gather(j)
            scatter(T - NR + j, j, rows[j])

        plsc.subcore_barrier()
        pltpu.sync_copy(acc.at[pl.ds(r0, rpt)], out_s.at[cid, pl.ds(r0, rpt)])

    return pl.kernel(body, out_type=out_type, mesh=mesh, scratch_types=scratch)


def _lrelu(v):
    return jnp.where(v >= 0.0, v, 0.01 * v)


def _mm(a, b):
    return jnp.dot(a.astype(jnp.bfloat16), b.astype(jnp.bfloat16),
                   preferred_element_type=jnp.float32)


def _bs(shape, imap):
    return pl.BlockSpec(shape, imap)


ROWBLK = 512

_ROWMAP = lambda i: (i, 0)
_PARTMAP = lambda i: (0, i, 0)
_FIXMAP = lambda i: (0, 0)

_SP128 = _bs((2, ROWBLK, 128), _PARTMAP)
_H = _bs((ROWBLK, 128), _ROWMAP)
_H16 = _bs((ROWBLK, 16), _ROWMAP)
_H64 = _bs((ROWBLK, 64), _ROWMAP)
_W128 = _bs((128, 128), _FIXMAP)
_W64 = _bs((128, 64), _FIXMAP)
_W64_128 = _bs((64, 128), _FIXMAP)
_W16 = _bs((128, 16), _FIXMAP)
_B128 = _bs((1, 128), _FIXMAP)
_B64 = _bs((1, 64), _FIXMAP)
_B16 = _bs((1, 16), _FIXMAP)

_GRID = (NP // ROWBLK,)


def _call(body, in_specs, out_specs, out_shapes):
    return pl.pallas_call(
        body,
        grid=_GRID,
        in_specs=in_specs,
        out_specs=out_specs,
        out_shape=out_shapes,
    )


def _l1_body(sp, cp, h, wl, bl, wr, o, dv):
    cnt = cp[0, :, 0:16] + cp[1, :, 0:16]
    dinv = 1.0 / jnp.maximum(cnt, 1.0)
    agg = (sp[0] + sp[1]) * dinv[:, 0:1]
    o[...] = jnp.tanh(_mm(agg, wl[...]) + bl[...] + _mm(h[...], wr[...]))
    dv[...] = dinv


def _mid_body(act, sp, dv, h, wl, bl, wr, o):
    agg = (sp[0] + sp[1]) * dv[:, 0:1]
    o[...] = act(_mm(agg, wl[...]) + bl[...] + _mm(h[...], wr[...]))


def _l3_body(sp, dv, h, wl, bl, wr, wt, bt, wd1, bd1, mu_o, o0_o):
    agg = (sp[0] + sp[1]) * dv[:, 0:1]
    h3 = jnp.tanh(_mm(agg, wl[...]) + bl[...] + _mm(h[...], wr[...]))
    mu = _mm(h3, wt[...]) + bt[...]
    mu_o[...] = mu
    o0_o[...] = _lrelu(_mm(mu, wd1[...]) + bd1[...])


def _l6_body(sp, dv, h, wl6, bl6, wr6, z_o):
    agg = (sp[0] + sp[1]) * dv[:, 0:1]
    z_o[...] = _mm(agg, wl6[...]) + bl6[...] + _mm(h[...], wr6[...])


def kernel(x, edge_index, Wl1, bl1, Wr1, Wl2, bl2, Wr2, Wl3, bl3, Wr3,
           Wt, bt, Wd1, bd1, Wl4, bl4, Wr4, Wl5, bl5, Wr5, Wl6, bl6, Wr6):
    n, d = x.shape
    n_edges = edge_index.shape[1]
    f32 = jnp.float32

    xp = jnp.zeros((NP, d), f32).at[:n].set(x)
    epad = jnp.full((2, EPAD - n_edges), NP - 1, jnp.int32)
    ei = jnp.concatenate([edge_index.astype(jnp.int32), epad], axis=1)
    srcv = ei[0]
    dstv = ei[1].reshape(NW, T, 1, CHUNK)
    z128 = jnp.zeros((NP, 128), f32)
    onesc = jnp.ones((CHUNK, 128), f32)
    bl1r, bl2r, bl3r = bl1.reshape(1, 128), bl2.reshape(1, 128), bl3.reshape(1, 128)
    bl4r, bl5r = bl4.reshape(1, 128), bl5.reshape(1, 128)
    btr, bd1r = bt.reshape(1, 64), bd1.reshape(1, 128)
    wl6p = jnp.zeros((128, 16), f32).at[:, :3].set(Wl6)
    wr6p = jnp.zeros((128, 16), f32).at[:, :3].set(Wr6)
    bl6p = jnp.zeros((1, 16), f32).at[0, :3].set(bl6)

    agg = _make_agg()

    s1p, c1p = _make_agg(with_cnt=True)(xp, srcv, dstv, z128, onesc)
    h1, dv = _call(
        _l1_body,
        [_SP128, _SP128, _H, _W128, _B128, _W128],
        [_H, _H16],
        [jax.ShapeDtypeStruct((NP, 128), f32), jax.ShapeDtypeStruct((NP, 16), f32)],
    )(s1p, c1p, xp, Wl1, bl1r, Wr1)

    (s2p,) = agg(h1, srcv, dstv, z128)
    h2 = _call(
        functools.partial(_mid_body, jnp.tanh),
        [_SP128, _H16, _H, _W128, _B128, _W128],
        _H,
        jax.ShapeDtypeStruct((NP, 128), f32),
    )(s2p, dv, h1, Wl2, bl2r, Wr2)

    (s3p,) = agg(h2, srcv, dstv, z128)
    mu, o0 = _call(
        _l3_body,
        [_SP128, _H16, _H, _W128, _B128, _W128, _W64, _B64, _W64_128, _B128],
        [_H64, _H],
        [jax.ShapeDtypeStruct((NP, 64), f32), jax.ShapeDtypeStruct((NP, 128), f32)],
    )(s3p, dv, h2, Wl3, bl3r, Wr3, Wt, btr, Wd1, bd1r)

    (s4p,) = agg(o0, srcv, dstv, z128)
    o1 = _call(
        functools.partial(_mid_body, _lrelu),
        [_SP128, _H16, _H, _W128, _B128, _W128],
        _H,
        jax.ShapeDtypeStruct((NP, 128), f32),
    )(s4p, dv, o0, Wl4, bl4r, Wr4)

    (s5p,) = agg(o1, srcv, dstv, z128)
    o2 = _call(
        functools.partial(_mid_body, jnp.tanh),
        [_SP128, _H16, _H, _W128, _B128, _W128],
        _H,
        jax.ShapeDtypeStruct((NP, 128), f32),
    )(s5p, dv, o1, Wl5, bl5r, Wr5)

    (s6p,) = agg(o2, srcv, dstv, z128)
    z2 = _call(
        _l6_body,
        [_SP128, _H16, _H, _W16, _B16, _W16],
        _H16,
        jax.ShapeDtypeStruct((NP, 16), f32),
    )(s6p, dv, o2, wl6p, bl6p, wr6p)

    mu_n = mu[:n]
    return (z2[:n, :2], z2[:n, 2], mu_n, mu_n)


# speedup vs baseline: 1.3462x; 1.1091x over previous
import functools

import jax
import jax.numpy as jnp
from jax import lax
from jax.experimental import pallas as pl
from jax.experimental.pallas import tpu as pltpu
from jax.experimental.pallas import tpu_sc as plsc

NP = 10240
NW = 32
CHUNK = 80
EPW = 125 * CHUNK
EPAD = NW * EPW
ROWBLK = 512


def _make_agg(with_cnt=False):
    iters = EPW // CHUNK
    rpt = NP // 16

    mesh = plsc.VectorSubcoreMesh(core_axis_name="c", subcore_axis_name="s")
    out_type = [jax.ShapeDtypeStruct((2, NP, 128), jnp.float32)]
    scratch = [
        pltpu.VMEM((CHUNK,), jnp.int32),
        pltpu.VMEM((CHUNK,), jnp.int32),
        pltpu.VMEM((CHUNK, 128), jnp.float32),
        pltpu.VMEM_SHARED((NP, 128), jnp.float32),
        pltpu.SemaphoreType.DMA,
    ]
    if with_cnt:
        out_type.append(jax.ShapeDtypeStruct((2, NP, 128), jnp.float32))
        scratch.append(pltpu.VMEM((CHUNK, 128), jnp.float32))

    def body(*refs):
        if with_cnt:
            (y_hbm, src_hbm, dst_hbm, zf_hbm, ones_hbm,
             out_s, out_c, scur, dcur, rows, acc, sem, ones_v) = refs
        else:
            (y_hbm, src_hbm, dst_hbm, zf_hbm,
             out_s, scur, dcur, rows, acc, sem) = refs
        cid = lax.axis_index("c")
        sid = lax.axis_index("s")
        wid = sid * 2 + cid
        r0 = sid * rpt
        base0 = wid * EPW

        def zero_acc():
            pltpu.sync_copy(zf_hbm.at[pl.ds(r0, rpt)], acc.at[pl.ds(r0, rpt)])

        if with_cnt:
            zero_acc()
            pltpu.sync_copy(ones_hbm, ones_v)
            plsc.subcore_barrier()

            def cstep(k, carry):
                pltpu.sync_copy(dst_hbm.at[pl.ds(base0 + k * CHUNK, CHUNK)], dcur)
                pltpu.sync_copy(ones_v, acc.at[dcur], add=True)
                return carry

            lax.fori_loop(0, iters, cstep, 0)
            plsc.subcore_barrier()
            pltpu.sync_copy(acc.at[pl.ds(r0, rpt)],
                            out_c.at[cid, pl.ds(r0, rpt)])

        zero_acc()
        plsc.subcore_barrier()

        def step(k, carry):
            pltpu.sync_copy(src_hbm.at[pl.ds(base0 + k * CHUNK, CHUNK)], scur)
            pltpu.sync_copy(dst_hbm.at[pl.ds(base0 + k * CHUNK, CHUNK)], dcur)
            pltpu.async_copy(y_hbm.at[scur], rows, sem).wait()
            pltpu.sync_copy(rows, acc.at[dcur], add=True)
            return carry

        lax.fori_loop(0, iters, step, 0)
        plsc.subcore_barrier()
        pltpu.sync_copy(acc.at[pl.ds(r0, rpt)], out_s.at[cid, pl.ds(r0, rpt)])

    return pl.kernel(body, out_type=out_type, mesh=mesh, scratch_types=scratch)


def _lrelu(v):
    return jnp.where(v >= 0.0, v, 0.01 * v)


def _mm(a, b):
    return jnp.dot(a.astype(jnp.bfloat16), b.astype(jnp.bfloat16),
                   preferred_element_type=jnp.float32)


def _bs(shape, imap):
    return pl.BlockSpec(shape, imap)


_ROWMAP = lambda i: (i, 0)
_PARTMAP = lambda i: (0, i, 0)
_FIXMAP = lambda i: (0, 0)

_SP128 = _bs((2, ROWBLK, 128), _PARTMAP)
_SP16 = _bs((2, ROWBLK, 16), _PARTMAP)
_SPC = _bs((2, ROWBLK, 128), _PARTMAP)
_H = _bs((ROWBLK, 128), _ROWMAP)
_H16 = _bs((ROWBLK, 16), _ROWMAP)
_H64 = _bs((ROWBLK, 64), _ROWMAP)
_W128 = _bs((128, 128), _FIXMAP)
_W64 = _bs((128, 64), _FIXMAP)
_W64_128 = _bs((64, 128), _FIXMAP)
_W16 = _bs((128, 16), _FIXMAP)
_B128 = _bs((1, 128), _FIXMAP)
_B64 = _bs((1, 64), _FIXMAP)
_B16 = _bs((1, 16), _FIXMAP)

_GRID = (NP // ROWBLK,)


def _call(body, in_specs, out_specs, out_shapes):
    return pl.pallas_call(
        body,
        grid=_GRID,
        in_specs=in_specs,
        out_specs=out_specs,
        out_shape=out_shapes,
    )


def _l1_body(sp, cp, h, wl, bl, wr, o, dv):
    cnt = cp[0, :, 0:16] + cp[1, :, 0:16]
    dinv = 1.0 / jnp.maximum(cnt, 1.0)
    agg = (sp[0] + sp[1]) * dinv[:, 0:1]
    o[...] = jnp.tanh(_mm(agg, wl[...]) + bl[...] + _mm(h[...], wr[...]))
    dv[...] = dinv


def _mid_body(act, sp, dv, h, wl, bl, wr, o):
    agg = (sp[0] + sp[1]) * dv[:, 0:1]
    o[...] = act(_mm(agg, wl[...]) + bl[...] + _mm(h[...], wr[...]))


def _l3_body(sp, dv, h, wl, bl, wr, wt, bt, wd1, bd1, mu_o, o0_o):
    agg = (sp[0] + sp[1]) * dv[:, 0:1]
    h3 = jnp.tanh(_mm(agg, wl[...]) + bl[...] + _mm(h[...], wr[...]))
    mu = _mm(h3, wt[...]) + bt[...]
    mu_o[...] = mu
    o0_o[...] = _lrelu(_mm(mu, wd1[...]) + bd1[...])


def _l6_body(sp, dv, h, wl6, bl6, wr6, z_o):
    agg = (sp[0] + sp[1]) * dv[:, 0:1]
    z_o[...] = _mm(agg, wl6[...]) + bl6[...] + _mm(h[...], wr6[...])


def kernel(x, edge_index, Wl1, bl1, Wr1, Wl2, bl2, Wr2, Wl3, bl3, Wr3,
           Wt, bt, Wd1, bd1, Wl4, bl4, Wr4, Wl5, bl5, Wr5, Wl6, bl6, Wr6):
    n, d = x.shape
    n_edges = edge_index.shape[1]
    f32 = jnp.float32

    xp = jnp.zeros((NP, d), f32).at[:n].set(x)
    src2d = edge_index[0].astype(jnp.int32)
    dst2d = edge_index[1].astype(jnp.int32)
    z128 = jnp.zeros((NP, 128), f32)
    onesc = jnp.ones((CHUNK, 128), f32)
    bl1r, bl2r, bl3r = bl1.reshape(1, 128), bl2.reshape(1, 128), bl3.reshape(1, 128)
    bl4r, bl5r = bl4.reshape(1, 128), bl5.reshape(1, 128)
    btr, bd1r = bt.reshape(1, 64), bd1.reshape(1, 128)
    wl6p = jnp.zeros((128, 16), f32).at[:, :3].set(Wl6)
    wr6p = jnp.zeros((128, 16), f32).at[:, :3].set(Wr6)
    bl6p = jnp.zeros((1, 16), f32).at[0, :3].set(bl6)

    agg = _make_agg()

    s1p, c1p = _make_agg(with_cnt=True)(xp, src2d, dst2d, z128, onesc)
    h1, dv = _call(
        _l1_body,
        [_SP128, _SPC, _H, _W128, _B128, _W128],
        [_H, _H16],
        [jax.ShapeDtypeStruct((NP, 128), f32), jax.ShapeDtypeStruct((NP, 16), f32)],
    )(s1p, c1p, xp, Wl1, bl1r, Wr1)

    (s2p,) = agg(h1, src2d, dst2d, z128)
    h2 = _call(
        functools.partial(_mid_body, jnp.tanh),
        [_SP128, _H16, _H, _W128, _B128, _W128],
        _H,
        jax.ShapeDtypeStruct((NP, 128), f32),
    )(s2p, dv, h1, Wl2, bl2r, Wr2)

    (s3p,) = agg(h2, src2d, dst2d, z128)
    mu, o0 = _call(
        _l3_body,
        [_SP128, _H16, _H, _W128, _B128, _W128, _W64, _B64, _W64_128, _B128],
        [_H64, _H],
        [jax.ShapeDtypeStruct((NP, 64), f32), jax.ShapeDtypeStruct((NP, 128), f32)],
    )(s3p, dv, h2, Wl3, bl3r, Wr3, Wt, btr, Wd1, bd1r)

    (s4p,) = agg(o0, src2d, dst2d, z128)
    o1 = _call(
        functools.partial(_mid_body, _lrelu),
        [_SP128, _H16, _H, _W128, _B128, _W128],
        _H,
        jax.ShapeDtypeStruct((NP, 128), f32),
    )(s4p, dv, o0, Wl4, bl4r, Wr4)

    (s5p,) = agg(o1, src2d, dst2d, z128)
    o2 = _call(
        functools.partial(_mid_body, jnp.tanh),
        [_SP128, _H16, _H, _W128, _B128, _W128],
        _H,
        jax.ShapeDtypeStruct((NP, 128), f32),
    )(s5p, dv, o1, Wl5, bl5r, Wr5)

    (s6p,) = agg(o2, src2d, dst2d, z128)
    z2 = _call(
        _l6_body,
        [_SP128, _H16, _H, _W16, _B16, _W16],
        _H16,
        jax.ShapeDtypeStruct((NP, 16), f32),
    )(s6p, dv, o2, wl6p, bl6p, wr6p)

    mu_n = mu[:n]
    return (z2[:n, :2], z2[:n, 2], mu_n, mu_n)
